# baseline (speedup 1.0000x reference)
import jax
import jax.numpy as jnp
from jax import lax
from jax.experimental import pallas as pl
from jax.experimental.pallas import tpu as pltpu
from jax.experimental.pallas import tpu_sc as plsc

_N = 10000
_E = 160000
_C = 128
_A = 10
_RB = 8
_AVG_NEIGH = 16.0

_NC = 2
_NS = 16
_NW = _NC * _NS
_CH = 128

_WAVES = ((61440, 15, 0, 30, 0),
          (61440, 15, 0, 30, 0),
          (37120, 9, 8, 18, 16))

_RPS = 624


def _h_body(nf_ref, wup_ref, h_ref):
    h_ref[...] = jnp.dot(nf_ref[...], wup_ref[...],
                         preferred_element_type=jnp.float32) * (1.0 / jnp.sqrt(128.0))


def _edge_body(ef_ref, ea_ref, xs_ref, w1_ref, w2_ref, w3_ref, w4_ref, m_ref):
    t = jax.nn.silu(jnp.dot(ef_ref[...], w1_ref[...],
                            preferred_element_type=jnp.float32) * (1.0 / jnp.sqrt(8.0)))
    t = jax.nn.silu(jnp.dot(t.astype(jnp.bfloat16), w2_ref[...].astype(jnp.bfloat16),
                            preferred_element_type=jnp.float32) * 0.125)
    t = jax.nn.silu(jnp.dot(t.astype(jnp.bfloat16), w3_ref[...].astype(jnp.bfloat16),
                            preferred_element_type=jnp.float32) * 0.125)
    tp = jnp.dot(t.astype(jnp.bfloat16), w4_ref[...].astype(jnp.bfloat16),
                 preferred_element_type=jnp.float32) * 0.125
    xs = xs_ref[...]
    ea = ea_ref[...]
    m_ref[0] = xs * ea[:, 0:1] * tp[:, :_C]
    a1 = xs * tp[:, _C:]
    m_ref[1] = a1 * ea[:, 1:2]
    m_ref[2] = a1 * ea[:, 2:3]
    m_ref[3] = a1 * ea[:, 3:4]


def _node_body(ma_ref, mb_ref, mc_ref, na_ref, wl0_ref, wl1_ref, wsk0_ref,
               wsk1_ref, o_ref):
    s_lin = 1.0 / jnp.sqrt(128.0) / _AVG_NEIGH
    n_sk = 1.0 / jnp.sqrt(128.0 * _A)
    na = na_ref[...]
    for k in range(4):
        wl = wl0_ref[...] if k == 0 else wl1_ref[...]
        wsk = wsk0_ref[...] if k == 0 else wsk1_ref[...]
        msum = ma_ref[k] + mb_ref[k] + mc_ref[k]
        mk = jnp.dot(msum.astype(jnp.bfloat16), wl,
                     preferred_element_type=jnp.float32) * s_lin
        tk = jnp.dot(mk.astype(jnp.bfloat16), wsk,
                     preferred_element_type=jnp.float32)
        acc = na[:, 0:1] * tk[:, :_C]
        for v in range(1, _A):
            acc = acc + na[:, v:v + 1] * tk[:, _C * v:_C * (v + 1)]
        o_ref[k] = acc * n_sk


def _make_gather_body(nfull, tail):
    epw = nfull * _CH + tail

    def body(h_hbm, snd_hbm, xs_hbm, *scratch):
        if tail:
            idx_v, idxt_v, rows_v, rowst_v, sem_i, sem_g, sem_w = scratch
        else:
            idx_v, rows_v, sem_i, sem_g, sem_w = scratch
        w = lax.axis_index("s") * _NC + lax.axis_index("c")
        base = w * epw
        _KG = 3

        pltpu.async_copy(snd_hbm.at[pl.ds(base, _CH)], idx_v.at[0], sem_i)

        def outer(g, carry):
            for b in range(_KG):
                j = g * _KG + b
                bm1 = (b + 2) % _KG
                bp1 = (b + 1) % _KG

                @pl.when(j >= 2)
                def _wait_wb():
                    pltpu.make_async_copy(
                        rows_v.at[bp1],
                        xs_hbm.at[pl.ds(base + (j - 2) * _CH, _CH)], sem_w).wait()

                @pl.when(j >= 1)
                def _wait_gather():
                    pltpu.make_async_copy(h_hbm.at[idx_v.at[bm1]],
                                          rows_v.at[bm1], sem_g).wait()

                @pl.when(j >= 1)
                def _fire_wb():
                    pltpu.async_copy(
                        rows_v.at[bm1],
                        xs_hbm.at[pl.ds(base + (j - 1) * _CH, _CH)], sem_w)

                @pl.when(j + 1 <= nfull - 1)
                def _fire_idx():
                    pltpu.async_copy(snd_hbm.at[pl.ds(base + (j + 1) * _CH, _CH)],
                                     idx_v.at[bp1], sem_i)

                pltpu.make_async_copy(snd_hbm.at[pl.ds(base + j * _CH, _CH)],
                                      idx_v.at[b], sem_i).wait()
                pltpu.async_copy(h_hbm.at[idx_v.at[b]], rows_v.at[b], sem_g)
            return carry

        lax.fori_loop(0, nfull // _KG, outer, 0)
        bl = (nfull - 1) % _KG
        pltpu.make_async_copy(h_hbm.at[idx_v.at[bl]], rows_v.at[bl], sem_g).wait()
        pltpu.async_copy(rows_v.at[bl],
                         xs_hbm.at[pl.ds(base + (nfull - 1) * _CH, _CH)], sem_w)
        for j in (nfull - 2, nfull - 1):
            pltpu.make_async_copy(
                rows_v.at[j % _KG],
                xs_hbm.at[pl.ds(base + j * _CH, _CH)], sem_w).wait()

        if tail:
            tb = base + nfull * _CH
            pltpu.sync_copy(snd_hbm.at[pl.ds(tb, tail)], idxt_v)
            pltpu.async_copy(h_hbm.at[idxt_v], rowst_v, sem_g).wait()
            pltpu.sync_copy(rowst_v, xs_hbm.at[pl.ds(tb, tail)])

    return body


def _make_scatter_body(sfull, tail):
    eps = sfull * _CH + tail

    def body(m_hbm, rcv_hbm, out_hbm, slab, idx_v, vals_v, sem_f, sem_s, sem_z):
        c = lax.axis_index("c")
        s = lax.axis_index("s")
        base = s * eps
        _KS = 3
        _ZT = _RPS - 4 * _CH

        for kk in range(2):
            sec = c * 2 + kk

            def zrow(i, carry):
                for t in range(_C // 16):
                    vals_v[2, i, pl.ds(t * 16, 16)] = jnp.zeros((16,), jnp.float32)
                return carry

            lax.fori_loop(0, _CH, zrow, 0)
            for t in range(4):
                pltpu.async_copy(vals_v.at[2],
                                 slab.at[pl.ds(s * _RPS + t * _CH, _CH)], sem_z)
            pltpu.async_copy(vals_v.at[2, pl.ds(0, _ZT)],
                             slab.at[pl.ds(s * _RPS + 4 * _CH, _ZT)], sem_z)

            @pl.when(s == _NS - 1)
            def _zero_tail():
                pltpu.async_copy(vals_v.at[2, pl.ds(0, 16)],
                                 slab.at[pl.ds(_NS * _RPS, 16)], sem_z)

            for t in range(4):
                pltpu.make_async_copy(vals_v.at[2],
                                      slab.at[pl.ds(s * _RPS, _CH)], sem_z).wait()
            pltpu.make_async_copy(vals_v.at[2, pl.ds(0, _ZT)],
                                  slab.at[pl.ds(s * _RPS, _ZT)], sem_z).wait()

            @pl.when(s == _NS - 1)
            def _zero_tail_wait():
                pltpu.make_async_copy(vals_v.at[2, pl.ds(0, 16)],
                                      slab.at[pl.ds(_NS * _RPS, 16)], sem_z).wait()

            plsc.subcore_barrier()

            for b in range(2):
                pltpu.async_copy(rcv_hbm.at[pl.ds(base + b * _CH, _CH)],
                                 idx_v.at[b], sem_f)
                pltpu.async_copy(m_hbm.at[sec, pl.ds(base + b * _CH, _CH)],
                                 vals_v.at[b], sem_f)

            def outer(g, carry):
                for b in range(_KS):
                    j = g * _KS + b
                    b2 = (b + 2) % _KS

                    pltpu.make_async_copy(rcv_hbm.at[pl.ds(base + j * _CH, _CH)],
                                          idx_v.at[b], sem_f).wait()
                    pltpu.make_async_copy(m_hbm.at[sec, pl.ds(base + j * _CH, _CH)],
                                          vals_v.at[b], sem_f).wait()
                    pltpu.async_copy(vals_v.at[b], slab.at[idx_v.at[b]], sem_s,
                                     add=True)

                    @pl.when(j >= 1)
                    def _wait_prev_scatter():
                        pltpu.make_async_copy(vals_v.at[b2],
                                              slab.at[idx_v.at[b2]], sem_s).wait()

                    @pl.when(j + 2 <= sfull - 1)
                    def _fire_fill():
                        pltpu.async_copy(
                            rcv_hbm.at[pl.ds(base + (j + 2) * _CH, _CH)],
                            idx_v.at[b2], sem_f)
                        pltpu.async_copy(
                            m_hbm.at[sec, pl.ds(base + (j + 2) * _CH, _CH)],
                            vals_v.at[b2], sem_f)
                return carry

            lax.fori_loop(0, sfull // _KS, outer, 0)
            pltpu.make_async_copy(vals_v.at[(sfull - 1) % _KS],
                                  slab.at[idx_v.at[(sfull - 1) % _KS]],
                                  sem_s).wait()
            if tail:
                pltpu.sync_copy(rcv_hbm.at[pl.ds(base + sfull * _CH, tail)],
                                idx_v.at[0, pl.ds(0, tail)])
                pltpu.sync_copy(m_hbm.at[sec, pl.ds(base + sfull * _CH, tail)],
                                vals_v.at[0, pl.ds(0, tail)])
                pltpu.sync_copy(vals_v.at[0, pl.ds(0, tail)],
                                slab.at[idx_v.at[0, pl.ds(0, tail)]], add=True)
            plsc.subcore_barrier()

            pltpu.sync_copy(slab.at[pl.ds(s * _RPS, _RPS)],
                            out_hbm.at[sec, pl.ds(s * _RPS, _RPS)])

            @pl.when(s == _NS - 1)
            def _write_tail():
                pltpu.sync_copy(slab.at[pl.ds(_NS * _RPS, 16)],
                                out_hbm.at[sec, pl.ds(_NS * _RPS, 16)])

    return body


def _gather_call(h, snd, ne, nfull, tail):
    scratch = [pltpu.VMEM((3, _CH), jnp.int32)]
    if tail:
        scratch.append(pltpu.VMEM((tail,), jnp.int32))
    scratch.append(pltpu.VMEM((3, _CH, _C), jnp.float32))
    if tail:
        scratch.append(pltpu.VMEM((tail, _C), jnp.float32))
    scratch += [pltpu.SemaphoreType.DMA] * 3
    return pl.kernel(
        _make_gather_body(nfull, tail),
        out_type=jax.ShapeDtypeStruct((ne, _C), jnp.float32),
        mesh=plsc.VectorSubcoreMesh(core_axis_name="c", subcore_axis_name="s"),
        scratch_types=scratch,
    )(h, snd)


def _edge_call(ef, ea, xs, W1, W2, W3, W4, ne, _BE):
    return pl.pallas_call(
        _edge_body,
        grid=(ne // _BE,),
        in_specs=[pl.BlockSpec((_BE, _RB), lambda i: (i, 0)),
                  pl.BlockSpec((_BE, 4), lambda i: (i, 0)),
                  pl.BlockSpec((_BE, _C), lambda i: (i, 0)),
                  pl.BlockSpec((_RB, 64), lambda i: (0, 0)),
                  pl.BlockSpec((64, 64), lambda i: (0, 0)),
                  pl.BlockSpec((64, 64), lambda i: (0, 0)),
                  pl.BlockSpec((64, 2 * _C), lambda i: (0, 0))],
        out_specs=pl.BlockSpec((4, _BE, _C), lambda i: (0, i, 0)),
        out_shape=jax.ShapeDtypeStruct((4, ne, _C), jnp.float32),
    )(ef, ea, xs, W1, W2, W3, W4)


def _scatter_call(m4, rcv, sfull, tail):
    return pl.kernel(
        _make_scatter_body(sfull, tail),
        out_type=jax.ShapeDtypeStruct((4, _N, _C), jnp.float32),
        mesh=plsc.VectorSubcoreMesh(core_axis_name="c", subcore_axis_name="s"),
        scratch_types=[
            pltpu.VMEM_SHARED((_N, _C), jnp.float32),
            pltpu.VMEM((3, _CH), jnp.int32),
            pltpu.VMEM((3, _CH, _C), jnp.float32),
            pltpu.SemaphoreType.DMA,
            pltpu.SemaphoreType.DMA,
            pltpu.SemaphoreType.DMA,
        ],
    )(m4, rcv)


def kernel(node_attrs, node_feats, edge_attrs, edge_feats, edge_index,
           W_up, W1, W2, W3, W4, W_lin0, W_lin1, W_sk0, W_sk1):
    sender = edge_index[0]
    recv = edge_index[1]

    h = pl.pallas_call(
        _h_body,
        grid=(10,),
        in_specs=[pl.BlockSpec((1000, _C), lambda i: (i, 0)),
                  pl.BlockSpec((_C, _C), lambda i: (0, 0))],
        out_specs=pl.BlockSpec((1000, _C), lambda i: (i, 0)),
        out_shape=jax.ShapeDtypeStruct((_N, _C), jnp.float32),
    )(node_feats, W_up)

    xs = []
    off = 0
    for (ne, gf, gt, sf, st) in _WAVES:
        xs.append(_gather_call(h, sender[off:off + ne], ne, gf, gt))
        off += ne

    msgs = []
    off = 0
    for w, (ne, gf, gt, sf, st) in enumerate(_WAVES):
        m4 = _edge_call(edge_feats[off:off + ne], edge_attrs[off:off + ne],
                        xs[w], W1, W2, W3, W4, ne, 5120 if w < 2 else 4640)
        msgs.append(_scatter_call(m4, recv[off:off + ne], sf, st))
        off += ne

    _BN = 2000
    o4 = pl.pallas_call(
        _node_body,
        grid=(_N // _BN,),
        in_specs=[pl.BlockSpec((4, _BN, _C), lambda i: (0, i, 0)),
                  pl.BlockSpec((4, _BN, _C), lambda i: (0, i, 0)),
                  pl.BlockSpec((4, _BN, _C), lambda i: (0, i, 0)),
                  pl.BlockSpec((_BN, _A), lambda i: (i, 0)),
                  pl.BlockSpec((_C, _C), lambda i: (0, 0)),
                  pl.BlockSpec((_C, _C), lambda i: (0, 0)),
                  pl.BlockSpec((_C, _A * _C), lambda i: (0, 0)),
                  pl.BlockSpec((_C, _A * _C), lambda i: (0, 0))],
        out_specs=pl.BlockSpec((4, _BN, _C), lambda i: (0, i, 0)),
        out_shape=jax.ShapeDtypeStruct((4, _N, _C), jnp.float32),
    )(msgs[0], msgs[1], msgs[2], node_attrs, W_lin0, W_lin1,
      W_sk0.reshape(_C, _A * _C), W_sk1.reshape(_C, _A * _C))

    return jnp.transpose(o4, (1, 2, 0))


# speedup vs baseline: 1.0025x; 1.0025x over previous
import jax
import jax.numpy as jnp
from jax import lax
from jax.experimental import pallas as pl
from jax.experimental.pallas import tpu as pltpu
from jax.experimental.pallas import tpu_sc as plsc

_N = 10000
_E = 160000
_C = 128
_A = 10
_RB = 8
_AVG_NEIGH = 16.0

_NC = 2
_NS = 16
_NW = _NC * _NS
_CH = 128

_WAVES = ((61440, 15, 0, 30, 0),
          (61440, 15, 0, 30, 0),
          (37120, 9, 8, 18, 16))

_RPS = 624


def _h_body(nf_ref, wup_ref, h_ref):
    h_ref[...] = jnp.dot(nf_ref[...], wup_ref[...],
                         preferred_element_type=jnp.float32) * (1.0 / jnp.sqrt(128.0))


def _edge_body(ef_ref, ea_ref, xs_ref, w1_ref, w2_ref, w3_ref, w4_ref, m_ref):
    t = jax.nn.silu(jnp.dot(ef_ref[...], w1_ref[...],
                            preferred_element_type=jnp.float32) * (1.0 / jnp.sqrt(8.0)))
    t = jax.nn.silu(jnp.dot(t.astype(jnp.bfloat16), w2_ref[...].astype(jnp.bfloat16),
                            preferred_element_type=jnp.float32) * 0.125)
    t = jax.nn.silu(jnp.dot(t.astype(jnp.bfloat16), w3_ref[...].astype(jnp.bfloat16),
                            preferred_element_type=jnp.float32) * 0.125)
    tp = jnp.dot(t.astype(jnp.bfloat16), w4_ref[...].astype(jnp.bfloat16),
                 preferred_element_type=jnp.float32) * 0.125
    xs = xs_ref[...]
    ea = ea_ref[...]
    m_ref[0] = xs * ea[:, 0:1] * tp[:, :_C]
    a1 = xs * tp[:, _C:]
    m_ref[1] = a1 * ea[:, 1:2]
    m_ref[2] = a1 * ea[:, 2:3]
    m_ref[3] = a1 * ea[:, 3:4]


def _node_body(ma_ref, mb_ref, mc_ref, na_ref, wl0_ref, wl1_ref, wsk0_ref,
               wsk1_ref, o_ref):
    s_lin = 1.0 / jnp.sqrt(128.0) / _AVG_NEIGH
    n_sk = 1.0 / jnp.sqrt(128.0 * _A)
    na = na_ref[...]
    for k in range(4):
        wl = wl0_ref[...] if k == 0 else wl1_ref[...]
        wsk = wsk0_ref[...] if k == 0 else wsk1_ref[...]
        msum = ma_ref[k] + mb_ref[k] + mc_ref[k]
        mk = jnp.dot(msum.astype(jnp.bfloat16), wl,
                     preferred_element_type=jnp.float32) * s_lin
        tk = jnp.dot(mk.astype(jnp.bfloat16), wsk,
                     preferred_element_type=jnp.float32)
        acc = na[:, 0:1] * tk[:, :_C]
        for v in range(1, _A):
            acc = acc + na[:, v:v + 1] * tk[:, _C * v:_C * (v + 1)]
        o_ref[k] = acc * n_sk


def _make_gather_body(nfull, tail):
    epw = nfull * _CH + tail

    def body(h_hbm, snd_hbm, xs_hbm, *scratch):
        if tail:
            idx_v, idxt_v, rows_v, rowst_v, sem_i, sem_g, sem_w = scratch
        else:
            idx_v, rows_v, sem_i, sem_g, sem_w = scratch
        w = lax.axis_index("s") * _NC + lax.axis_index("c")
        base = w * epw
        _KG = 3

        pltpu.async_copy(snd_hbm.at[pl.ds(base, _CH)], idx_v.at[0], sem_i)

        def outer(g, carry):
            for b in range(_KG):
                j = g * _KG + b
                bm1 = (b + 2) % _KG
                bp1 = (b + 1) % _KG

                @pl.when(j >= 2)
                def _wait_wb():
                    pltpu.make_async_copy(
                        rows_v.at[bp1],
                        xs_hbm.at[pl.ds(base + (j - 2) * _CH, _CH)], sem_w).wait()

                @pl.when(j >= 1)
                def _wait_gather():
                    pltpu.make_async_copy(h_hbm.at[idx_v.at[bm1]],
                                          rows_v.at[bm1], sem_g).wait()

                @pl.when(j >= 1)
                def _fire_wb():
                    pltpu.async_copy(
                        rows_v.at[bm1],
                        xs_hbm.at[pl.ds(base + (j - 1) * _CH, _CH)], sem_w)

                @pl.when(j + 1 <= nfull - 1)
                def _fire_idx():
                    pltpu.async_copy(snd_hbm.at[pl.ds(base + (j + 1) * _CH, _CH)],
                                     idx_v.at[bp1], sem_i)

                pltpu.make_async_copy(snd_hbm.at[pl.ds(base + j * _CH, _CH)],
                                      idx_v.at[b], sem_i).wait()
                pltpu.async_copy(h_hbm.at[idx_v.at[b]], rows_v.at[b], sem_g)
            return carry

        lax.fori_loop(0, nfull // _KG, outer, 0)
        bl = (nfull - 1) % _KG
        pltpu.make_async_copy(h_hbm.at[idx_v.at[bl]], rows_v.at[bl], sem_g).wait()
        pltpu.async_copy(rows_v.at[bl],
                         xs_hbm.at[pl.ds(base + (nfull - 1) * _CH, _CH)], sem_w)
        for j in (nfull - 2, nfull - 1):
            pltpu.make_async_copy(
                rows_v.at[j % _KG],
                xs_hbm.at[pl.ds(base + j * _CH, _CH)], sem_w).wait()

        if tail:
            tb = base + nfull * _CH
            pltpu.sync_copy(snd_hbm.at[pl.ds(tb, tail)], idxt_v)
            pltpu.async_copy(h_hbm.at[idxt_v], rowst_v, sem_g).wait()
            pltpu.sync_copy(rowst_v, xs_hbm.at[pl.ds(tb, tail)])

    return body


def _make_scatter_body(sfull, tail):
    eps = sfull * _CH + tail

    def body(m_hbm, rcv_hbm, out_hbm, slab, idx_v, vals_v, sem_f, sem_s, sem_z):
        c = lax.axis_index("c")
        s = lax.axis_index("s")
        base = s * eps
        _KS = 3
        _ZT = _RPS - 4 * _CH

        for kk in range(2):
            sec = c * 2 + kk

            def zrow(i, carry):
                for t in range(_C // 16):
                    vals_v[2, i, pl.ds(t * 16, 16)] = jnp.zeros((16,), jnp.float32)
                return carry

            lax.fori_loop(0, _CH, zrow, 0)
            for t in range(4):
                pltpu.async_copy(vals_v.at[2],
                                 slab.at[pl.ds(s * _RPS + t * _CH, _CH)], sem_z)
            pltpu.async_copy(vals_v.at[2, pl.ds(0, _ZT)],
                             slab.at[pl.ds(s * _RPS + 4 * _CH, _ZT)], sem_z)

            @pl.when(s == _NS - 1)
            def _zero_tail():
                pltpu.async_copy(vals_v.at[2, pl.ds(0, 16)],
                                 slab.at[pl.ds(_NS * _RPS, 16)], sem_z)

            for t in range(4):
                pltpu.make_async_copy(vals_v.at[2],
                                      slab.at[pl.ds(s * _RPS, _CH)], sem_z).wait()
            pltpu.make_async_copy(vals_v.at[2, pl.ds(0, _ZT)],
                                  slab.at[pl.ds(s * _RPS, _ZT)], sem_z).wait()

            @pl.when(s == _NS - 1)
            def _zero_tail_wait():
                pltpu.make_async_copy(vals_v.at[2, pl.ds(0, 16)],
                                      slab.at[pl.ds(_NS * _RPS, 16)], sem_z).wait()

            plsc.subcore_barrier()

            for b in range(2):
                pltpu.async_copy(rcv_hbm.at[pl.ds(base + b * _CH, _CH)],
                                 idx_v.at[b], sem_f)
                pltpu.async_copy(m_hbm.at[sec, pl.ds(base + b * _CH, _CH)],
                                 vals_v.at[b], sem_f)

            def outer(g, carry):
                for b in range(_KS):
                    j = g * _KS + b
                    b2 = (b + 2) % _KS

                    pltpu.make_async_copy(rcv_hbm.at[pl.ds(base + j * _CH, _CH)],
                                          idx_v.at[b], sem_f).wait()
                    pltpu.make_async_copy(m_hbm.at[sec, pl.ds(base + j * _CH, _CH)],
                                          vals_v.at[b], sem_f).wait()
                    pltpu.async_copy(vals_v.at[b], slab.at[idx_v.at[b]], sem_s,
                                     add=True)

                    @pl.when(j >= 1)
                    def _wait_prev_scatter():
                        pltpu.make_async_copy(vals_v.at[b2],
                                              slab.at[idx_v.at[b2]], sem_s).wait()

                    @pl.when(j + 2 <= sfull - 1)
                    def _fire_fill():
                        pltpu.async_copy(
                            rcv_hbm.at[pl.ds(base + (j + 2) * _CH, _CH)],
                            idx_v.at[b2], sem_f)
                        pltpu.async_copy(
                            m_hbm.at[sec, pl.ds(base + (j + 2) * _CH, _CH)],
                            vals_v.at[b2], sem_f)
                return carry

            lax.fori_loop(0, sfull // _KS, outer, 0)
            pltpu.make_async_copy(vals_v.at[(sfull - 1) % _KS],
                                  slab.at[idx_v.at[(sfull - 1) % _KS]],
                                  sem_s).wait()
            if tail:
                pltpu.sync_copy(rcv_hbm.at[pl.ds(base + sfull * _CH, tail)],
                                idx_v.at[0, pl.ds(0, tail)])
                pltpu.sync_copy(m_hbm.at[sec, pl.ds(base + sfull * _CH, tail)],
                                vals_v.at[0, pl.ds(0, tail)])
                pltpu.sync_copy(vals_v.at[0, pl.ds(0, tail)],
                                slab.at[idx_v.at[0, pl.ds(0, tail)]], add=True)
            plsc.subcore_barrier()

            pltpu.sync_copy(slab.at[pl.ds(s * _RPS, _RPS)],
                            out_hbm.at[sec, pl.ds(s * _RPS, _RPS)])

            @pl.when(s == _NS - 1)
            def _write_tail():
                pltpu.sync_copy(slab.at[pl.ds(_NS * _RPS, 16)],
                                out_hbm.at[sec, pl.ds(_NS * _RPS, 16)])

    return body


def _gather_call(h, snd, ne, nfull, tail):
    scratch = [pltpu.VMEM((3, _CH), jnp.int32)]
    if tail:
        scratch.append(pltpu.VMEM((tail,), jnp.int32))
    scratch.append(pltpu.VMEM((3, _CH, _C), jnp.float32))
    if tail:
        scratch.append(pltpu.VMEM((tail, _C), jnp.float32))
    scratch += [pltpu.SemaphoreType.DMA] * 3
    return pl.kernel(
        _make_gather_body(nfull, tail),
        out_type=jax.ShapeDtypeStruct((ne, _C), jnp.float32),
        mesh=plsc.VectorSubcoreMesh(core_axis_name="c", subcore_axis_name="s"),
        scratch_types=scratch,
    )(h, snd)


def _edge_call(ef, ea, xs, W1, W2, W3, W4, ne, _BE):
    return pl.pallas_call(
        _edge_body,
        grid=(ne // _BE,),
        in_specs=[pl.BlockSpec((_BE, _RB), lambda i: (i, 0)),
                  pl.BlockSpec((_BE, 4), lambda i: (i, 0)),
                  pl.BlockSpec((_BE, _C), lambda i: (i, 0)),
                  pl.BlockSpec((_RB, 64), lambda i: (0, 0)),
                  pl.BlockSpec((64, 64), lambda i: (0, 0)),
                  pl.BlockSpec((64, 64), lambda i: (0, 0)),
                  pl.BlockSpec((64, 2 * _C), lambda i: (0, 0))],
        out_specs=pl.BlockSpec((4, _BE, _C), lambda i: (0, i, 0)),
        out_shape=jax.ShapeDtypeStruct((4, ne, _C), jnp.float32),
    )(ef, ea, xs, W1, W2, W3, W4)


def _scatter_call(m4, rcv, sfull, tail):
    return pl.kernel(
        _make_scatter_body(sfull, tail),
        out_type=jax.ShapeDtypeStruct((4, _N, _C), jnp.float32),
        mesh=plsc.VectorSubcoreMesh(core_axis_name="c", subcore_axis_name="s"),
        scratch_types=[
            pltpu.VMEM_SHARED((_N, _C), jnp.float32),
            pltpu.VMEM((3, _CH), jnp.int32),
            pltpu.VMEM((3, _CH, _C), jnp.float32),
            pltpu.SemaphoreType.DMA,
            pltpu.SemaphoreType.DMA,
            pltpu.SemaphoreType.DMA,
        ],
    )(m4, rcv)


def kernel(node_attrs, node_feats, edge_attrs, edge_feats, edge_index,
           W_up, W1, W2, W3, W4, W_lin0, W_lin1, W_sk0, W_sk1):
    sender = edge_index[0]
    recv = edge_index[1]

    h = pl.pallas_call(
        _h_body,
        grid=(10,),
        in_specs=[pl.BlockSpec((1000, _C), lambda i: (i, 0)),
                  pl.BlockSpec((_C, _C), lambda i: (0, 0))],
        out_specs=pl.BlockSpec((1000, _C), lambda i: (i, 0)),
        out_shape=jax.ShapeDtypeStruct((_N, _C), jnp.float32),
    )(node_feats, W_up)

    xs = []
    off = 0
    for (ne, gf, gt, sf, st) in _WAVES:
        xs.append(_gather_call(h, sender[off:off + ne], ne, gf, gt))
        off += ne

    msgs = []
    off = 0
    for w, (ne, gf, gt, sf, st) in enumerate(_WAVES):
        m4 = _edge_call(edge_feats[off:off + ne], edge_attrs[off:off + ne],
                        xs[w], W1, W2, W3, W4, ne, 5120 if w < 2 else 4640)
        msgs.append(_scatter_call(m4, recv[off:off + ne], sf, st))
        off += ne

    _BN = 1000
    o4 = pl.pallas_call(
        _node_body,
        grid=(_N // _BN,),
        in_specs=[pl.BlockSpec((4, _BN, _C), lambda i: (0, i, 0)),
                  pl.BlockSpec((4, _BN, _C), lambda i: (0, i, 0)),
                  pl.BlockSpec((4, _BN, _C), lambda i: (0, i, 0)),
                  pl.BlockSpec((_BN, _A), lambda i: (i, 0)),
                  pl.BlockSpec((_C, _C), lambda i: (0, 0)),
                  pl.BlockSpec((_C, _C), lambda i: (0, 0)),
                  pl.BlockSpec((_C, _A * _C), lambda i: (0, 0)),
                  pl.BlockSpec((_C, _A * _C), lambda i: (0, 0))],
        out_specs=pl.BlockSpec((4, _BN, _C), lambda i: (0, i, 0)),
        out_shape=jax.ShapeDtypeStruct((4, _N, _C), jnp.float32),
    )(msgs[0], msgs[1], msgs[2], node_attrs, W_lin0, W_lin1,
      W_sk0.reshape(_C, _A * _C), W_sk1.reshape(_C, _A * _C))

    return jnp.transpose(o4, (1, 2, 0))
